# baseline (speedup 1.0000x reference)
import jax
import jax.numpy as jnp
from jax import lax
from jax.experimental import pallas as pl
from jax.experimental.pallas import tpu as pltpu
from jax.experimental.pallas import tpu_sc as plsc

B = 16384
EMB = 16
CB_SIZE = 32
CB_DIM = 8

_NC = 2
_NS = 16
_NW = _NC * _NS
_BPW = B // _NW


def _sc_gather_body(item_t, brand_t, cate_t, idx_i, idx_b, idx_c,
                    out_i, out_b, out_c,
                    iv0, iv1, iv2, s0, s1, s2):
    wid = lax.axis_index("s") * _NC + lax.axis_index("c")
    base = wid * _BPW
    pltpu.sync_copy(idx_i.at[pl.ds(base, _BPW)], iv0)
    pltpu.sync_copy(idx_b.at[pl.ds(base, _BPW)], iv1)
    pltpu.sync_copy(idx_c.at[pl.ds(base, _BPW)], iv2)

    def chunk(c, do_start):
        b16 = c * 16
        v0 = iv0[pl.ds(b16, 16)]
        v1 = iv1[pl.ds(b16, 16)]
        v2 = iv2[pl.ds(b16, 16)]
        for j in range(16):
            row = base + b16 + j
            c0 = pltpu.make_async_copy(item_t.at[v0[j]], out_i.at[row], s0)
            c1 = pltpu.make_async_copy(brand_t.at[v1[j]], out_b.at[row], s1)
            c2 = pltpu.make_async_copy(cate_t.at[v2[j]], out_c.at[row], s2)
            if do_start:
                c0.start(); c1.start(); c2.start()
            else:
                c0.wait(); c1.wait(); c2.wait()

    def start_loop(c, _):
        chunk(c, True)
        return 0

    def wait_loop(c, _):
        chunk(c, False)
        return 0

    lax.fori_loop(0, _BPW // 16, start_loop, 0)
    lax.fori_loop(0, _BPW // 16, wait_loop, 0)


@jax.jit
def _sc_gather(item_t, brand_t, cate_t, idx_i, idx_b, idx_c):
    mesh = plsc.VectorSubcoreMesh(core_axis_name="c", subcore_axis_name="s")
    row = jax.ShapeDtypeStruct((B, EMB), jnp.float32)
    run = pl.kernel(
        _sc_gather_body,
        mesh=mesh,
        out_type=(row, row, row),
        scratch_types=[
            pltpu.VMEM((_BPW,), jnp.int32),
            pltpu.VMEM((_BPW,), jnp.int32),
            pltpu.VMEM((_BPW,), jnp.int32),
            pltpu.SemaphoreType.DMA,
            pltpu.SemaphoreType.DMA,
            pltpu.SemaphoreType.DMA,
        ],
    )
    return run(item_t, brand_t, cate_t, idx_i, idx_b, idx_c)


_BLK = 2048
_NBLK = B // _BLK


def _dense_body(item_emb_ref, brand_ref, cate_ref, count_ref, noise_ref,
                cb_ref, cbt_ref,
                me_w1, me_b1, me_w2, me_b2,
                lv_w1, lv_b1, lv_w2, lv_b2,
                mp_w1, mp_b1, mp_w2, mp_b2,
                lp_w1, lp_b1, lp_w2, lp_b2,
                dec_w1, dec_b1, dec_w2, dec_b2,
                head_w, head_b,
                recon_ref, reg_ref, target_ref, vq_ref, perp_ref,
                counts_ref):
    f32 = jnp.float32
    step = pl.program_id(0)
    item = item_emb_ref[...]
    brand = brand_ref[...]
    cate = cate_ref[...]
    noise = noise_ref[...]
    count = count_ref[...]

    def mlp2(x, w1, b1, w2, b2):
        h = jnp.maximum(jnp.dot(x, w1[...], preferred_element_type=f32)
                        + b1[...], 0.0)
        return jnp.dot(h, w2[...], preferred_element_type=f32) + b2[...]

    mean = mlp2(item, me_w1, me_b1, me_w2, me_b2)
    log_v = mlp2(item, lv_w1, lv_b1, lv_w2, lv_b2)

    def mlp2_side(w1, b1, w2, b2):
        h = (jnp.dot(brand, w1[:EMB, :], preferred_element_type=f32)
             + jnp.dot(cate, w1[EMB:, :], preferred_element_type=f32)
             + b1[...])
        h = jnp.maximum(h, 0.0)
        return jnp.dot(h, w2[...], preferred_element_type=f32) + b2[...]

    mean_p = mlp2_side(mp_w1, mp_b1, mp_w2, mp_b2)
    log_v_p = mlp2_side(lp_w1, lp_b1, lp_w2, lp_b2)

    p1 = jnp.sum(jnp.square(mean - mean_p), axis=1)
    p2 = jnp.sum(jnp.square(jnp.exp(log_v * 0.5) - jnp.exp(log_v_p * 0.5)),
                 axis=1)
    reg_part = jnp.reshape(jnp.sum(p1 + p2), (1, 1))

    z = mean + jnp.exp(log_v * 0.5) * noise

    cb = cb_ref[...]
    cbt = cbt_ref[...]
    cb2 = jnp.sum(cbt * cbt, axis=0, keepdims=True)
    score = cb2 - 2.0 * jnp.dot(z, cbt, preferred_element_type=f32)
    m = jnp.min(score, axis=1, keepdims=True)
    iota = lax.broadcasted_iota(jnp.int32, score.shape, 1)
    idx = jnp.min(jnp.where(score == m, iota, CB_SIZE), axis=1,
                  keepdims=True)
    one_hot = (iota == idx).astype(f32)
    z_q = jnp.dot(one_hot, cb, preferred_element_type=f32)

    vq_part = jnp.reshape(jnp.sum(jnp.square(z_q - z)), (1, 1))
    counts_part = jnp.reshape(jnp.sum(one_hot, axis=0), (1, CB_SIZE))

    h = (jnp.dot(z, dec_w1[:CB_DIM, :], preferred_element_type=f32)
         + count * dec_w1[CB_DIM:, :][...].reshape(1, -1) + dec_b1[...])
    h = jnp.maximum(h, 0.0)
    pred = jnp.dot(h, dec_w2[...], preferred_element_type=f32) + dec_b2[...]

    recon_part = jnp.reshape(jnp.sum(jnp.square(pred - item)), (1, 1))

    logit = jnp.dot(pred, head_w[...], preferred_element_type=f32) + head_b[...]
    target_ref[...] = 1.0 / (1.0 + jnp.exp(-logit))

    @pl.when(step == 0)
    def _init():
        reg_ref[...] = reg_part
        vq_ref[...] = vq_part
        recon_ref[...] = recon_part
        counts_ref[...] = counts_part

    @pl.when(step > 0)
    def _acc():
        reg_ref[...] += reg_part
        vq_ref[...] += vq_part
        recon_ref[...] += recon_part
        counts_ref[...] += counts_part

    @pl.when(step == _NBLK - 1)
    def _finalize():
        vq_ref[...] = vq_ref[...] * (1.25 / (B * CB_DIM))
        recon_ref[...] = recon_ref[...] * (1.0 / B)
        probs = counts_ref[...] * (1.0 / B)
        ent = jnp.sum(probs * jnp.log(probs + 1e-10))
        perp_ref[...] = jnp.reshape(jnp.exp(-ent), (1, 1))


@jax.jit
def _dense(item_emb, brand_emb, cate_emb, count, noise, codebook,
           me_w1, me_b1, me_w2, me_b2,
           lv_w1, lv_b1, lv_w2, lv_b2,
           mp_w1, mp_b1, mp_w2, mp_b2,
           lp_w1, lp_b1, lp_w2, lp_b2,
           dec_w1, dec_b1, dec_w2, dec_b2,
           head_w, head_b):
    scalar = jax.ShapeDtypeStruct((1, 1), jnp.float32)
    out_shape = (scalar, scalar,
                 jax.ShapeDtypeStruct((B, 1), jnp.float32),
                 scalar, scalar)
    blk = lambda i: (i, 0)
    cst2 = lambda i: (0, 0)
    cst1 = lambda i: (0,)
    w2spec = pl.BlockSpec(index_map=cst2)
    w1spec = pl.BlockSpec(index_map=cst1)
    row_spec = lambda w: pl.BlockSpec((_BLK, w), blk)
    scal_spec = pl.BlockSpec((1, 1), cst2)
    wspecs = [w2spec, w1spec, w2spec, w1spec,
              w2spec, w1spec, w2spec, w1spec,
              w2spec, w1spec, w2spec, w1spec,
              w2spec, w1spec, w2spec, w1spec,
              w2spec, w1spec, w2spec, w1spec,
              w2spec, w1spec]
    outs = pl.pallas_call(
        _dense_body,
        grid=(_NBLK,),
        in_specs=[row_spec(EMB), row_spec(EMB), row_spec(EMB),
                  row_spec(1), row_spec(CB_DIM), w2spec, w2spec] + wspecs,
        out_specs=(scal_spec, scal_spec, pl.BlockSpec((_BLK, 1), blk),
                   scal_spec, scal_spec),
        scratch_shapes=[pltpu.VMEM((1, CB_SIZE), jnp.float32)],
        out_shape=out_shape,
    )(item_emb, brand_emb, cate_emb, count, noise, codebook, codebook.T,
      me_w1, me_b1, me_w2, me_b2,
      lv_w1, lv_b1, lv_w2, lv_b2,
      mp_w1, mp_b1, mp_w2, mp_b2,
      lp_w1, lp_b1, lp_w2, lp_b2,
      dec_w1, dec_b1, dec_w2, dec_b2,
      head_w, head_b)
    recon, reg, target, vq, perp = outs
    return (recon[0, 0], reg[0, 0], target, vq[0, 0], perp[0, 0])


def kernel(item_id, feat_brand, feat_cate, count, noise, item_emb_table,
           brand_table, cate_table, codebook,
           me_w1, me_b1, me_w2, me_b2,
           lv_w1, lv_b1, lv_w2, lv_b2,
           mp_w1, mp_b1, mp_w2, mp_b2,
           lp_w1, lp_b1, lp_w2, lp_b2,
           dec_w1, dec_b1, dec_w2, dec_b2,
           head_w, head_b):
    item_emb, brand_emb, cate_emb = _sc_gather(
        item_emb_table, brand_table, cate_table,
        item_id.astype(jnp.int32), feat_brand.astype(jnp.int32),
        feat_cate.astype(jnp.int32))
    return _dense(item_emb, brand_emb, cate_emb, count, noise, codebook,
                  me_w1, me_b1, me_w2, me_b2,
                  lv_w1, lv_b1, lv_w2, lv_b2,
                  mp_w1, mp_b1, mp_w2, mp_b2,
                  lp_w1, lp_b1, lp_w2, lp_b2,
                  dec_w1, dec_b1, dec_w2, dec_b2,
                  head_w, head_b)


# speedup vs baseline: 4.7287x; 4.7287x over previous
import jax
import jax.numpy as jnp
from jax import lax
from jax.experimental import pallas as pl
from jax.experimental.pallas import tpu as pltpu
from jax.experimental.pallas import tpu_sc as plsc

B = 16384
EMB = 16
CB_SIZE = 32
CB_DIM = 8

_NC = 2
_NS = 16
_NW = _NC * _NS
_BPW = B // _NW


_QPW = _BPW // 8


def _sc_gather_body(item_t, brand_t, cate_t, idx_i, idx_b, idx_c,
                    out_i, out_b, out_c,
                    iv0, iv1, iv2, rv0, rv1, rv2, s0, s1, s2):
    wid = lax.axis_index("s") * _NC + lax.axis_index("c")
    base = wid * _BPW
    pltpu.sync_copy(idx_i.at[pl.ds(base, _BPW)], iv0)
    pltpu.sync_copy(idx_b.at[pl.ds(base, _BPW)], iv1)
    pltpu.sync_copy(idx_c.at[pl.ds(base, _BPW)], iv2)
    c0 = pltpu.async_copy(item_t.at[iv0], rv0, s0)
    c1 = pltpu.async_copy(brand_t.at[iv1], rv1, s1)
    c2 = pltpu.async_copy(cate_t.at[iv2], rv2, s2)
    c0.wait()
    c1.wait()
    c2.wait()

    qb = wid * _QPW

    def wb(k, do_start):
        g0 = pltpu.make_async_copy(
            rv0.at[pl.ds(8 * k, 8)], out_i.at[qb + k, :, pl.ds(0, EMB)], s0)
        g1 = pltpu.make_async_copy(
            rv1.at[pl.ds(8 * k, 8)], out_b.at[qb + k, :, pl.ds(0, EMB)], s1)
        g2 = pltpu.make_async_copy(
            rv2.at[pl.ds(8 * k, 8)], out_c.at[qb + k, :, pl.ds(0, EMB)], s2)
        if do_start:
            g0.start(); g1.start(); g2.start()
        else:
            g0.wait(); g1.wait(); g2.wait()

    def wb_start(k, _):
        wb(k, True)
        return 0

    def wb_wait(k, _):
        wb(k, False)
        return 0

    lax.fori_loop(0, _QPW, wb_start, 0)
    lax.fori_loop(0, _QPW, wb_wait, 0)


@jax.jit
def _sc_gather(item_t, brand_t, cate_t, idx_i, idx_b, idx_c):
    mesh = plsc.VectorSubcoreMesh(core_axis_name="c", subcore_axis_name="s")
    row = jax.ShapeDtypeStruct((B // 8, 8, 128), jnp.float32)
    run = pl.kernel(
        _sc_gather_body,
        mesh=mesh,
        compiler_params=pltpu.CompilerParams(use_tc_tiling_on_sc=False),
        out_type=(row, row, row),
        scratch_types=[
            pltpu.VMEM((_BPW,), jnp.int32),
            pltpu.VMEM((_BPW,), jnp.int32),
            pltpu.VMEM((_BPW,), jnp.int32),
            pltpu.VMEM((_BPW, EMB), jnp.float32),
            pltpu.VMEM((_BPW, EMB), jnp.float32),
            pltpu.VMEM((_BPW, EMB), jnp.float32),
            pltpu.SemaphoreType.DMA,
            pltpu.SemaphoreType.DMA,
            pltpu.SemaphoreType.DMA,
        ],
    )
    return run(item_t, brand_t, cate_t, idx_i, idx_b, idx_c)


_BLK = 2048
_NBLK = B // _BLK


def _dense_body(item_emb_ref, brand_ref, cate_ref, count_ref, noise_ref,
                cb_ref, cbt_ref,
                me_w1, me_b1, me_w2, me_b2,
                lv_w1, lv_b1, lv_w2, lv_b2,
                mp_w1, mp_b1, mp_w2, mp_b2,
                lp_w1, lp_b1, lp_w2, lp_b2,
                dec_w1, dec_b1, dec_w2, dec_b2,
                head_w, head_b,
                recon_ref, reg_ref, target_ref, vq_ref, perp_ref,
                counts_ref):
    f32 = jnp.float32
    step = pl.program_id(0)
    item = item_emb_ref[:, :, pl.ds(0, EMB)].reshape(_BLK, EMB)
    brand = brand_ref[:, :, pl.ds(0, EMB)].reshape(_BLK, EMB)
    cate = cate_ref[:, :, pl.ds(0, EMB)].reshape(_BLK, EMB)
    noise = noise_ref[...]
    count = count_ref[...]

    def mlp2(x, w1, b1, w2, b2):
        h = jnp.maximum(jnp.dot(x, w1[...], preferred_element_type=f32)
                        + b1[...], 0.0)
        return jnp.dot(h, w2[...], preferred_element_type=f32) + b2[...]

    mean = mlp2(item, me_w1, me_b1, me_w2, me_b2)
    log_v = mlp2(item, lv_w1, lv_b1, lv_w2, lv_b2)

    def mlp2_side(w1, b1, w2, b2):
        h = (jnp.dot(brand, w1[:EMB, :], preferred_element_type=f32)
             + jnp.dot(cate, w1[EMB:, :], preferred_element_type=f32)
             + b1[...])
        h = jnp.maximum(h, 0.0)
        return jnp.dot(h, w2[...], preferred_element_type=f32) + b2[...]

    mean_p = mlp2_side(mp_w1, mp_b1, mp_w2, mp_b2)
    log_v_p = mlp2_side(lp_w1, lp_b1, lp_w2, lp_b2)

    p1 = jnp.sum(jnp.square(mean - mean_p), axis=1)
    p2 = jnp.sum(jnp.square(jnp.exp(log_v * 0.5) - jnp.exp(log_v_p * 0.5)),
                 axis=1)
    reg_part = jnp.reshape(jnp.sum(p1 + p2), (1, 1))

    z = mean + jnp.exp(log_v * 0.5) * noise

    cb = cb_ref[...]
    cbt = cbt_ref[...]
    cb2 = jnp.sum(cbt * cbt, axis=0, keepdims=True)
    score = cb2 - 2.0 * jnp.dot(z, cbt, preferred_element_type=f32)
    m = jnp.min(score, axis=1, keepdims=True)
    iota = lax.broadcasted_iota(jnp.int32, score.shape, 1)
    idx = jnp.min(jnp.where(score == m, iota, CB_SIZE), axis=1,
                  keepdims=True)
    one_hot = (iota == idx).astype(f32)
    z_q = jnp.dot(one_hot, cb, preferred_element_type=f32)

    vq_part = jnp.reshape(jnp.sum(jnp.square(z_q - z)), (1, 1))
    counts_part = jnp.reshape(jnp.sum(one_hot, axis=0), (1, CB_SIZE))

    h = (jnp.dot(z, dec_w1[:CB_DIM, :], preferred_element_type=f32)
         + count * dec_w1[CB_DIM:, :][...].reshape(1, -1) + dec_b1[...])
    h = jnp.maximum(h, 0.0)
    pred = jnp.dot(h, dec_w2[...], preferred_element_type=f32) + dec_b2[...]

    recon_part = jnp.reshape(jnp.sum(jnp.square(pred - item)), (1, 1))

    logit = jnp.dot(pred, head_w[...], preferred_element_type=f32) + head_b[...]
    target_ref[...] = 1.0 / (1.0 + jnp.exp(-logit))

    @pl.when(step == 0)
    def _init():
        reg_ref[...] = reg_part
        vq_ref[...] = vq_part
        recon_ref[...] = recon_part
        counts_ref[...] = counts_part

    @pl.when(step > 0)
    def _acc():
        reg_ref[...] += reg_part
        vq_ref[...] += vq_part
        recon_ref[...] += recon_part
        counts_ref[...] += counts_part

    @pl.when(step == _NBLK - 1)
    def _finalize():
        vq_ref[...] = vq_ref[...] * (1.25 / (B * CB_DIM))
        recon_ref[...] = recon_ref[...] * (1.0 / B)
        probs = counts_ref[...] * (1.0 / B)
        ent = jnp.sum(probs * jnp.log(probs + 1e-10))
        perp_ref[...] = jnp.reshape(jnp.exp(-ent), (1, 1))


@jax.jit
def _dense(item_emb, brand_emb, cate_emb, count, noise, codebook,
           me_w1, me_b1, me_w2, me_b2,
           lv_w1, lv_b1, lv_w2, lv_b2,
           mp_w1, mp_b1, mp_w2, mp_b2,
           lp_w1, lp_b1, lp_w2, lp_b2,
           dec_w1, dec_b1, dec_w2, dec_b2,
           head_w, head_b):
    scalar = jax.ShapeDtypeStruct((1, 1), jnp.float32)
    out_shape = (scalar, scalar,
                 jax.ShapeDtypeStruct((B, 1), jnp.float32),
                 scalar, scalar)
    blk = lambda i: (i, 0)
    cst2 = lambda i: (0, 0)
    cst1 = lambda i: (0,)
    w2spec = pl.BlockSpec(index_map=cst2)
    w1spec = pl.BlockSpec(index_map=cst1)
    row_spec = lambda w: pl.BlockSpec((_BLK, w), blk)
    emb_spec = pl.BlockSpec((_BLK // 8, 8, 128), lambda i: (i, 0, 0))
    scal_spec = pl.BlockSpec((1, 1), cst2)
    wspecs = [w2spec, w1spec, w2spec, w1spec,
              w2spec, w1spec, w2spec, w1spec,
              w2spec, w1spec, w2spec, w1spec,
              w2spec, w1spec, w2spec, w1spec,
              w2spec, w1spec, w2spec, w1spec,
              w2spec, w1spec]
    outs = pl.pallas_call(
        _dense_body,
        grid=(_NBLK,),
        in_specs=[emb_spec, emb_spec, emb_spec,
                  row_spec(1), row_spec(CB_DIM), w2spec, w2spec] + wspecs,
        out_specs=(scal_spec, scal_spec, pl.BlockSpec((_BLK, 1), blk),
                   scal_spec, scal_spec),
        scratch_shapes=[pltpu.VMEM((1, CB_SIZE), jnp.float32)],
        out_shape=out_shape,
    )(item_emb, brand_emb, cate_emb, count, noise, codebook, codebook.T,
      me_w1, me_b1, me_w2, me_b2,
      lv_w1, lv_b1, lv_w2, lv_b2,
      mp_w1, mp_b1, mp_w2, mp_b2,
      lp_w1, lp_b1, lp_w2, lp_b2,
      dec_w1, dec_b1, dec_w2, dec_b2,
      head_w, head_b)
    recon, reg, target, vq, perp = outs
    return (recon[0, 0], reg[0, 0], target, vq[0, 0], perp[0, 0])


def kernel(item_id, feat_brand, feat_cate, count, noise, item_emb_table,
           brand_table, cate_table, codebook,
           me_w1, me_b1, me_w2, me_b2,
           lv_w1, lv_b1, lv_w2, lv_b2,
           mp_w1, mp_b1, mp_w2, mp_b2,
           lp_w1, lp_b1, lp_w2, lp_b2,
           dec_w1, dec_b1, dec_w2, dec_b2,
           head_w, head_b):
    item_emb, brand_emb, cate_emb = _sc_gather(
        item_emb_table, brand_table, cate_table,
        item_id.astype(jnp.int32), feat_brand.astype(jnp.int32),
        feat_cate.astype(jnp.int32))
    return _dense(item_emb, brand_emb, cate_emb, count, noise, codebook,
                  me_w1, me_b1, me_w2, me_b2,
                  lv_w1, lv_b1, lv_w2, lv_b2,
                  mp_w1, mp_b1, mp_w2, mp_b2,
                  lp_w1, lp_b1, lp_w2, lp_b2,
                  dec_w1, dec_b1, dec_w2, dec_b2,
                  head_w, head_b)


# speedup vs baseline: 9.1619x; 1.9375x over previous
import jax
import jax.numpy as jnp
from jax import lax
from jax.experimental import pallas as pl
from jax.experimental.pallas import tpu as pltpu
from jax.experimental.pallas import tpu_sc as plsc

B = 16384
V = 100000
EMB = 16
CB_SIZE = 32
CB_DIM = 8

_NC = 2
_NS = 16
_NW = _NC * _NS
_BPW = B // _NW
_GPW = _BPW * EMB


def _sc_gather_body(item_t, brand_t, cate_t, idx_i, idx_b, idx_c,
                    out_i, out_b, out_c,
                    iv0, iv1, iv2, gi0, gi1, gi2, gb0, gb1, gb2,
                    s0, s1, s2):
    wid = lax.axis_index("s") * _NC + lax.axis_index("c")
    base = wid * _BPW
    pltpu.sync_copy(idx_i.at[pl.ds(base, _BPW)], iv0)
    pltpu.sync_copy(idx_b.at[pl.ds(base, _BPW)], iv1)
    pltpu.sync_copy(idx_c.at[pl.ds(base, _BPW)], iv2)

    def expand(c, _):
        b16 = c * 16
        v0 = iv0[pl.ds(b16, 16)]
        v1 = iv1[pl.ds(b16, 16)]
        v2 = iv2[pl.ds(b16, 16)]
        for d in range(EMB):
            off = jnp.full((16,), d * V, jnp.int32)
            gi0[pl.ds(d * _BPW + b16, 16)] = v0 + off
            gi1[pl.ds(d * _BPW + b16, 16)] = v1 + off
            gi2[pl.ds(d * _BPW + b16, 16)] = v2 + off
        return 0

    lax.fori_loop(0, _BPW // 16, expand, 0)

    c0 = pltpu.async_copy(item_t.at[gi0], gb0, s0)
    c1 = pltpu.async_copy(brand_t.at[gi1], gb1, s1)
    c2 = pltpu.async_copy(cate_t.at[gi2], gb2, s2)
    c0.wait()
    c1.wait()
    c2.wait()

    def wb(d, do_start):
        g0 = pltpu.make_async_copy(gb0.at[pl.ds(d * _BPW, _BPW)],
                                   out_i.at[d, pl.ds(base, _BPW)], s0)
        g1 = pltpu.make_async_copy(gb1.at[pl.ds(d * _BPW, _BPW)],
                                   out_b.at[d, pl.ds(base, _BPW)], s1)
        g2 = pltpu.make_async_copy(gb2.at[pl.ds(d * _BPW, _BPW)],
                                   out_c.at[d, pl.ds(base, _BPW)], s2)
        if do_start:
            g0.start(); g1.start(); g2.start()
        else:
            g0.wait(); g1.wait(); g2.wait()

    for d in range(EMB):
        wb(d, True)
    for d in range(EMB):
        wb(d, False)


@jax.jit
def _sc_gather(item_t, brand_t, cate_t, idx_i, idx_b, idx_c):
    mesh = plsc.VectorSubcoreMesh(core_axis_name="c", subcore_axis_name="s")
    out = jax.ShapeDtypeStruct((EMB, B), jnp.float32)
    run = pl.kernel(
        _sc_gather_body,
        mesh=mesh,
        compiler_params=pltpu.CompilerParams(use_tc_tiling_on_sc=False),
        out_type=(out, out, out),
        scratch_types=[
            pltpu.VMEM((_BPW,), jnp.int32),
            pltpu.VMEM((_BPW,), jnp.int32),
            pltpu.VMEM((_BPW,), jnp.int32),
            pltpu.VMEM((_GPW,), jnp.int32),
            pltpu.VMEM((_GPW,), jnp.int32),
            pltpu.VMEM((_GPW,), jnp.int32),
            pltpu.VMEM((_GPW,), jnp.float32),
            pltpu.VMEM((_GPW,), jnp.float32),
            pltpu.VMEM((_GPW,), jnp.float32),
            pltpu.SemaphoreType.DMA,
            pltpu.SemaphoreType.DMA,
            pltpu.SemaphoreType.DMA,
        ],
    )
    return run(item_t, brand_t, cate_t, idx_i, idx_b, idx_c)


_BLK = 2048
_NBLK = B // _BLK


def _dense_body(item_ref, brand_ref, cate_ref, count_ref, noise_ref,
                cb_ref,
                me_w1, me_b1, me_w2, me_b2,
                lv_w1, lv_b1, lv_w2, lv_b2,
                mp_w1, mp_b1, mp_w2, mp_b2,
                lp_w1, lp_b1, lp_w2, lp_b2,
                dec_w1, dec_b1, dec_w2, dec_b2,
                head_w, head_b,
                recon_ref, reg_ref, target_ref, vq_ref, perp_ref,
                counts_ref):
    f32 = jnp.float32
    step = pl.program_id(0)
    item = item_ref[...]
    brand = brand_ref[...]
    cate = cate_ref[...]
    noise = noise_ref[...]
    count = count_ref[...]

    def tmat(w, x):
        return lax.dot_general(w[...], x, (((0,), (0,)), ((), ())),
                               preferred_element_type=f32)

    def col(b):
        return b[...].reshape(-1, 1)

    def mlp2(x, w1, b1, w2, b2):
        h = jnp.maximum(tmat(w1, x) + col(b1), 0.0)
        return tmat(w2, h) + col(b2)

    mean = mlp2(item, me_w1, me_b1, me_w2, me_b2)
    log_v = mlp2(item, lv_w1, lv_b1, lv_w2, lv_b2)

    def mlp2_side(w1, b1, w2, b2):
        h = (lax.dot_general(w1[pl.ds(0, EMB), :], brand,
                             (((0,), (0,)), ((), ())),
                             preferred_element_type=f32)
             + lax.dot_general(w1[pl.ds(EMB, EMB), :], cate,
                               (((0,), (0,)), ((), ())),
                               preferred_element_type=f32)
             + col(b1))
        h = jnp.maximum(h, 0.0)
        return tmat(w2, h) + col(b2)

    mean_p = mlp2_side(mp_w1, mp_b1, mp_w2, mp_b2)
    log_v_p = mlp2_side(lp_w1, lp_b1, lp_w2, lp_b2)

    p1 = jnp.sum(jnp.square(mean - mean_p), axis=0)
    p2 = jnp.sum(jnp.square(jnp.exp(log_v * 0.5) - jnp.exp(log_v_p * 0.5)),
                 axis=0)
    reg_part = jnp.reshape(jnp.sum(p1 + p2), (1, 1))

    z = mean + jnp.exp(log_v * 0.5) * noise

    cb = cb_ref[...]
    cb2 = jnp.sum(cb * cb, axis=1).reshape(-1, 1)
    score = cb2 - 2.0 * jnp.dot(cb, z, preferred_element_type=f32)
    m = jnp.min(score, axis=0, keepdims=True)
    iota = lax.broadcasted_iota(jnp.int32, score.shape, 0)
    idx = jnp.min(jnp.where(score == m, iota, CB_SIZE), axis=0,
                  keepdims=True)
    one_hot = (iota == idx).astype(f32)
    z_q = tmat(cb_ref, one_hot)

    vq_part = jnp.reshape(jnp.sum(jnp.square(z_q - z)), (1, 1))
    counts_part = jnp.sum(one_hot, axis=1).reshape(-1, 1)

    h = (tmat(dec_w1.at[pl.ds(0, CB_DIM), :], z)
         + col(dec_w1.at[CB_DIM]) * count + col(dec_b1))
    h = jnp.maximum(h, 0.0)
    pred = tmat(dec_w2, h) + col(dec_b2)

    recon_part = jnp.reshape(jnp.sum(jnp.square(pred - item)), (1, 1))

    logit = tmat(head_w, pred) + col(head_b)
    target_ref[...] = 1.0 / (1.0 + jnp.exp(-logit))

    @pl.when(step == 0)
    def _init():
        reg_ref[...] = reg_part
        vq_ref[...] = vq_part
        recon_ref[...] = recon_part
        counts_ref[...] = counts_part

    @pl.when(step > 0)
    def _acc():
        reg_ref[...] += reg_part
        vq_ref[...] += vq_part
        recon_ref[...] += recon_part
        counts_ref[...] += counts_part

    @pl.when(step == _NBLK - 1)
    def _finalize():
        vq_ref[...] = vq_ref[...] * (1.25 / (B * CB_DIM))
        recon_ref[...] = recon_ref[...] * (1.0 / B)
        probs = counts_ref[...] * (1.0 / B)
        ent = jnp.sum(probs * jnp.log(probs + 1e-10))
        perp_ref[...] = jnp.reshape(jnp.exp(-ent), (1, 1))


@jax.jit
def _dense(item_t, brand_t, cate_t, count_t, noise_t, codebook,
           me_w1, me_b1, me_w2, me_b2,
           lv_w1, lv_b1, lv_w2, lv_b2,
           mp_w1, mp_b1, mp_w2, mp_b2,
           lp_w1, lp_b1, lp_w2, lp_b2,
           dec_w1, dec_b1, dec_w2, dec_b2,
           head_w, head_b):
    scalar = jax.ShapeDtypeStruct((1, 1), jnp.float32)
    out_shape = (scalar, scalar,
                 jax.ShapeDtypeStruct((1, B), jnp.float32),
                 scalar, scalar)
    blk = lambda i: (0, i)
    cst2 = lambda i: (0, 0)
    cst1 = lambda i: (0,)
    w2spec = pl.BlockSpec(index_map=cst2)
    w1spec = pl.BlockSpec(index_map=cst1)
    tr_spec = lambda h: pl.BlockSpec((h, _BLK), blk)
    scal_spec = pl.BlockSpec((1, 1), cst2)
    wspecs = [w2spec, w1spec, w2spec, w1spec,
              w2spec, w1spec, w2spec, w1spec,
              w2spec, w1spec, w2spec, w1spec,
              w2spec, w1spec, w2spec, w1spec,
              w2spec, w1spec, w2spec, w1spec,
              w2spec, w1spec]
    outs = pl.pallas_call(
        _dense_body,
        grid=(_NBLK,),
        in_specs=[tr_spec(EMB), tr_spec(EMB), tr_spec(EMB),
                  tr_spec(1), tr_spec(CB_DIM), w2spec] + wspecs,
        out_specs=(scal_spec, scal_spec, pl.BlockSpec((1, _BLK), blk),
                   scal_spec, scal_spec),
        scratch_shapes=[pltpu.VMEM((CB_SIZE, 1), jnp.float32)],
        out_shape=out_shape,
    )(item_t, brand_t, cate_t, count_t, noise_t, codebook,
      me_w1, me_b1, me_w2, me_b2,
      lv_w1, lv_b1, lv_w2, lv_b2,
      mp_w1, mp_b1, mp_w2, mp_b2,
      lp_w1, lp_b1, lp_w2, lp_b2,
      dec_w1, dec_b1, dec_w2, dec_b2,
      head_w, head_b)
    recon, reg, target, vq, perp = outs
    return (recon[0, 0], reg[0, 0], target.reshape(B, 1),
            vq[0, 0], perp[0, 0])


def kernel(item_id, feat_brand, feat_cate, count, noise, item_emb_table,
           brand_table, cate_table, codebook,
           me_w1, me_b1, me_w2, me_b2,
           lv_w1, lv_b1, lv_w2, lv_b2,
           mp_w1, mp_b1, mp_w2, mp_b2,
           lp_w1, lp_b1, lp_w2, lp_b2,
           dec_w1, dec_b1, dec_w2, dec_b2,
           head_w, head_b):
    item_t, brand_t, cate_t = _sc_gather(
        item_emb_table.T.reshape(-1), brand_table.T.reshape(-1),
        cate_table.T.reshape(-1),
        item_id.astype(jnp.int32), feat_brand.astype(jnp.int32),
        feat_cate.astype(jnp.int32))
    return _dense(item_t, brand_t, cate_t,
                  count.reshape(1, B), noise.T, codebook,
                  me_w1, me_b1, me_w2, me_b2,
                  lv_w1, lv_b1, lv_w2, lv_b2,
                  mp_w1, mp_b1, mp_w2, mp_b2,
                  lp_w1, lp_b1, lp_w2, lp_b2,
                  dec_w1, dec_b1, dec_w2, dec_b2,
                  head_w, head_b)


# speedup vs baseline: 9.9029x; 1.0809x over previous
import jax
import jax.numpy as jnp
from jax import lax
from jax.experimental import pallas as pl
from jax.experimental.pallas import tpu as pltpu
from jax.experimental.pallas import tpu_sc as plsc

B = 16384
V = 100000
EMB = 16
CB_SIZE = 32
CB_DIM = 8

_NC = 2
_NS = 16
_NW = _NC * _NS
_BPW = B // _NW
_GPW = _BPW * EMB


def _sc_gather_body(item_t, brand_t, cate_t, idx_i, idx_b, idx_c,
                    out_i, out_b, out_c,
                    iv0, iv1, iv2, gi0, gi1, gi2, gb0, gb1, gb2,
                    s0, s1, s2):
    wid = lax.axis_index("s") * _NC + lax.axis_index("c")
    base = wid * _BPW
    pltpu.sync_copy(idx_i.at[pl.ds(base, _BPW)], iv0)
    pltpu.sync_copy(idx_b.at[pl.ds(base, _BPW)], iv1)
    pltpu.sync_copy(idx_c.at[pl.ds(base, _BPW)], iv2)

    def expand(c, _):
        b16 = c * 16
        v0 = iv0[pl.ds(b16, 16)]
        v1 = iv1[pl.ds(b16, 16)]
        v2 = iv2[pl.ds(b16, 16)]
        for d in range(EMB):
            off = jnp.full((16,), d * V, jnp.int32)
            gi0[pl.ds(d * _BPW + b16, 16)] = v0 + off
            gi1[pl.ds(d * _BPW + b16, 16)] = v1 + off
            gi2[pl.ds(d * _BPW + b16, 16)] = v2 + off
        return 0

    lax.fori_loop(0, _BPW // 16, expand, 0)

    c0 = pltpu.async_copy(item_t.at[gi0], gb0, s0)
    c1 = pltpu.async_copy(brand_t.at[gi1], gb1, s1)
    c2 = pltpu.async_copy(cate_t.at[gi2], gb2, s2)
    c0.wait()
    c1.wait()
    c2.wait()

    def wb(d, do_start):
        g0 = pltpu.make_async_copy(gb0.at[pl.ds(d * _BPW, _BPW)],
                                   out_i.at[d, pl.ds(base, _BPW)], s0)
        g1 = pltpu.make_async_copy(gb1.at[pl.ds(d * _BPW, _BPW)],
                                   out_b.at[d, pl.ds(base, _BPW)], s1)
        g2 = pltpu.make_async_copy(gb2.at[pl.ds(d * _BPW, _BPW)],
                                   out_c.at[d, pl.ds(base, _BPW)], s2)
        if do_start:
            g0.start(); g1.start(); g2.start()
        else:
            g0.wait(); g1.wait(); g2.wait()

    for d in range(EMB):
        wb(d, True)
    for d in range(EMB):
        wb(d, False)


@jax.jit
def _sc_gather(item_t, brand_t, cate_t, idx_i, idx_b, idx_c):
    mesh = plsc.VectorSubcoreMesh(core_axis_name="c", subcore_axis_name="s")
    out = jax.ShapeDtypeStruct((EMB, B), jnp.float32)
    run = pl.kernel(
        _sc_gather_body,
        mesh=mesh,
        compiler_params=pltpu.CompilerParams(use_tc_tiling_on_sc=False),
        out_type=(out, out, out),
        scratch_types=[
            pltpu.VMEM((_BPW,), jnp.int32),
            pltpu.VMEM((_BPW,), jnp.int32),
            pltpu.VMEM((_BPW,), jnp.int32),
            pltpu.VMEM((_GPW,), jnp.int32),
            pltpu.VMEM((_GPW,), jnp.int32),
            pltpu.VMEM((_GPW,), jnp.int32),
            pltpu.VMEM((_GPW,), jnp.float32),
            pltpu.VMEM((_GPW,), jnp.float32),
            pltpu.VMEM((_GPW,), jnp.float32),
            pltpu.SemaphoreType.DMA,
            pltpu.SemaphoreType.DMA,
            pltpu.SemaphoreType.DMA,
        ],
    )
    return run(item_t, brand_t, cate_t, idx_i, idx_b, idx_c)


_BLK = 16384
_NBLK = B // _BLK


def _dense_body(item_ref, brand_ref, cate_ref, count_ref, noise_ref,
                cb_ref,
                me_w1, me_b1, me_w2, me_b2,
                lv_w1, lv_b1, lv_w2, lv_b2,
                mp_w1, mp_b1, mp_w2, mp_b2,
                lp_w1, lp_b1, lp_w2, lp_b2,
                dec_w1, dec_b1, dec_w2, dec_b2,
                head_w, head_b,
                recon_ref, reg_ref, target_ref, vq_ref, perp_ref,
                counts_ref):
    f32 = jnp.float32
    step = pl.program_id(0)
    item = item_ref[...]
    brand = brand_ref[...]
    cate = cate_ref[...]
    noise = noise_ref[...]
    count = count_ref[...]

    def tmat(w, x):
        return lax.dot_general(w[...], x, (((0,), (0,)), ((), ())),
                               preferred_element_type=f32)

    def col(b):
        return b[...].reshape(-1, 1)

    def mlp2(x, w1, b1, w2, b2):
        h = jnp.maximum(tmat(w1, x) + col(b1), 0.0)
        return tmat(w2, h) + col(b2)

    mean = mlp2(item, me_w1, me_b1, me_w2, me_b2)
    log_v = mlp2(item, lv_w1, lv_b1, lv_w2, lv_b2)

    def mlp2_side(w1, b1, w2, b2):
        h = (lax.dot_general(w1[pl.ds(0, EMB), :], brand,
                             (((0,), (0,)), ((), ())),
                             preferred_element_type=f32)
             + lax.dot_general(w1[pl.ds(EMB, EMB), :], cate,
                               (((0,), (0,)), ((), ())),
                               preferred_element_type=f32)
             + col(b1))
        h = jnp.maximum(h, 0.0)
        return tmat(w2, h) + col(b2)

    mean_p = mlp2_side(mp_w1, mp_b1, mp_w2, mp_b2)
    log_v_p = mlp2_side(lp_w1, lp_b1, lp_w2, lp_b2)

    p1 = jnp.sum(jnp.square(mean - mean_p), axis=0)
    p2 = jnp.sum(jnp.square(jnp.exp(log_v * 0.5) - jnp.exp(log_v_p * 0.5)),
                 axis=0)
    reg_part = jnp.reshape(jnp.sum(p1 + p2), (1, 1))

    z = mean + jnp.exp(log_v * 0.5) * noise

    cb = cb_ref[...]
    cb2 = jnp.sum(cb * cb, axis=1).reshape(-1, 1)
    score = cb2 - 2.0 * jnp.dot(cb, z, preferred_element_type=f32)
    m = jnp.min(score, axis=0, keepdims=True)
    iota = lax.broadcasted_iota(jnp.int32, score.shape, 0)
    idx = jnp.min(jnp.where(score == m, iota, CB_SIZE), axis=0,
                  keepdims=True)
    one_hot = (iota == idx).astype(f32)
    z_q = tmat(cb_ref, one_hot)

    vq_part = jnp.reshape(jnp.sum(jnp.square(z_q - z)), (1, 1))
    counts_part = jnp.sum(one_hot, axis=1).reshape(-1, 1)

    h = (tmat(dec_w1.at[pl.ds(0, CB_DIM), :], z)
         + col(dec_w1.at[CB_DIM]) * count + col(dec_b1))
    h = jnp.maximum(h, 0.0)
    pred = tmat(dec_w2, h) + col(dec_b2)

    recon_part = jnp.reshape(jnp.sum(jnp.square(pred - item)), (1, 1))

    logit = tmat(head_w, pred) + col(head_b)
    target_ref[...] = 1.0 / (1.0 + jnp.exp(-logit))

    @pl.when(step == 0)
    def _init():
        reg_ref[...] = reg_part
        vq_ref[...] = vq_part
        recon_ref[...] = recon_part
        counts_ref[...] = counts_part

    @pl.when(step > 0)
    def _acc():
        reg_ref[...] += reg_part
        vq_ref[...] += vq_part
        recon_ref[...] += recon_part
        counts_ref[...] += counts_part

    @pl.when(step == _NBLK - 1)
    def _finalize():
        vq_ref[...] = vq_ref[...] * (1.25 / (B * CB_DIM))
        recon_ref[...] = recon_ref[...] * (1.0 / B)
        probs = counts_ref[...] * (1.0 / B)
        ent = jnp.sum(probs * jnp.log(probs + 1e-10))
        perp_ref[...] = jnp.reshape(jnp.exp(-ent), (1, 1))


@jax.jit
def _dense(item_t, brand_t, cate_t, count_t, noise_t, codebook,
           me_w1, me_b1, me_w2, me_b2,
           lv_w1, lv_b1, lv_w2, lv_b2,
           mp_w1, mp_b1, mp_w2, mp_b2,
           lp_w1, lp_b1, lp_w2, lp_b2,
           dec_w1, dec_b1, dec_w2, dec_b2,
           head_w, head_b):
    scalar = jax.ShapeDtypeStruct((1, 1), jnp.float32)
    out_shape = (scalar, scalar,
                 jax.ShapeDtypeStruct((1, B), jnp.float32),
                 scalar, scalar)
    blk = lambda i: (0, i)
    cst2 = lambda i: (0, 0)
    cst1 = lambda i: (0,)
    w2spec = pl.BlockSpec(index_map=cst2)
    w1spec = pl.BlockSpec(index_map=cst1)
    tr_spec = lambda h: pl.BlockSpec((h, _BLK), blk)
    scal_spec = pl.BlockSpec((1, 1), cst2)
    wspecs = [w2spec, w1spec, w2spec, w1spec,
              w2spec, w1spec, w2spec, w1spec,
              w2spec, w1spec, w2spec, w1spec,
              w2spec, w1spec, w2spec, w1spec,
              w2spec, w1spec, w2spec, w1spec,
              w2spec, w1spec]
    outs = pl.pallas_call(
        _dense_body,
        grid=(_NBLK,),
        in_specs=[tr_spec(EMB), tr_spec(EMB), tr_spec(EMB),
                  tr_spec(1), tr_spec(CB_DIM), w2spec] + wspecs,
        out_specs=(scal_spec, scal_spec, pl.BlockSpec((1, _BLK), blk),
                   scal_spec, scal_spec),
        scratch_shapes=[pltpu.VMEM((CB_SIZE, 1), jnp.float32)],
        out_shape=out_shape,
    )(item_t, brand_t, cate_t, count_t, noise_t, codebook,
      me_w1, me_b1, me_w2, me_b2,
      lv_w1, lv_b1, lv_w2, lv_b2,
      mp_w1, mp_b1, mp_w2, mp_b2,
      lp_w1, lp_b1, lp_w2, lp_b2,
      dec_w1, dec_b1, dec_w2, dec_b2,
      head_w, head_b)
    recon, reg, target, vq, perp = outs
    return (recon[0, 0], reg[0, 0], target.reshape(B, 1),
            vq[0, 0], perp[0, 0])


def kernel(item_id, feat_brand, feat_cate, count, noise, item_emb_table,
           brand_table, cate_table, codebook,
           me_w1, me_b1, me_w2, me_b2,
           lv_w1, lv_b1, lv_w2, lv_b2,
           mp_w1, mp_b1, mp_w2, mp_b2,
           lp_w1, lp_b1, lp_w2, lp_b2,
           dec_w1, dec_b1, dec_w2, dec_b2,
           head_w, head_b):
    item_t, brand_t, cate_t = _sc_gather(
        item_emb_table.T.reshape(-1), brand_table.T.reshape(-1),
        cate_table.T.reshape(-1),
        item_id.astype(jnp.int32), feat_brand.astype(jnp.int32),
        feat_cate.astype(jnp.int32))
    return _dense(item_t, brand_t, cate_t,
                  count.reshape(1, B), noise.T, codebook,
                  me_w1, me_b1, me_w2, me_b2,
                  lv_w1, lv_b1, lv_w2, lv_b2,
                  mp_w1, mp_b1, mp_w2, mp_b2,
                  lp_w1, lp_b1, lp_w2, lp_b2,
                  dec_w1, dec_b1, dec_w2, dec_b2,
                  head_w, head_b)
